# Initial kernel scaffold; baseline (speedup 1.0000x reference)
import jax
import jax.numpy as jnp
from jax.experimental import pallas as pl


def kernel(x, table):
    raise NotImplementedError("write your pallas kernel here")


# speedup vs baseline: 1.0765x; 1.0765x over previous
import functools
import math

import jax
import jax.numpy as jnp
import numpy as np
from jax import lax
from jax.experimental import pallas as pl
from jax.experimental.pallas import tpu as pltpu
from jax.experimental.pallas import tpu_sc as plsc

D_MODEL = 128
SEQ_LEN = 2048
BATCH = 4
SCALE = math.sqrt(float(D_MODEL))

NUM_CORES = 2
NUM_SUBCORES = 16
NUM_WORKERS = NUM_CORES * NUM_SUBCORES
ROWS_TOTAL = BATCH * SEQ_LEN
ROWS_PER_W = ROWS_TOTAL // NUM_WORKERS
GATHER_CHUNK = 128
N_CHUNKS = ROWS_PER_W // GATHER_CHUNK
WORKERS_PER_SEQ = SEQ_LEN // ROWS_PER_W
LANES = 16


def _positional_encoding_over_scale() -> np.ndarray:
    half = D_MODEL / 2
    positions = np.arange(SEQ_LEN)[:, np.newaxis]
    depths = np.arange(int(half))[np.newaxis, :] / half
    angle_rates = 1 / 10000 ** depths
    angle_rads = positions * angle_rates
    pe = np.concatenate([np.sin(angle_rads), np.cos(angle_rads)], axis=-1)
    return (pe / SCALE).astype(np.float32)


_POS_OVER_SCALE = jnp.asarray(_positional_encoding_over_scale())


def _emb_body(x_hbm, pos_hbm, table_hbm, out_hbm, idx_v, rows_v, sem):
    wid = lax.axis_index("s") * NUM_CORES + lax.axis_index("c")
    base = wid * ROWS_PER_W
    pstart = (wid % WORKERS_PER_SEQ) * ROWS_PER_W

    pltpu.sync_copy(x_hbm.at[pl.ds(wid * N_CHUNKS, N_CHUNKS)], idx_v)
    pltpu.sync_copy(pos_hbm.at[pl.ds(pstart, ROWS_PER_W)], rows_v)

    copies = []
    for j in range(N_CHUNKS):
        copies.append(pltpu.async_copy(
            table_hbm.at[idx_v.at[j]],
            rows_v.at[pl.ds(j * GATHER_CHUNK, GATHER_CHUNK)],
            sem, add=True))
    for c in copies:
        c.wait()

    def row_body(i, carry):
        for c in range(D_MODEL // LANES):
            sl = pl.ds(c * LANES, LANES)
            rows_v[i, sl] = rows_v[i, sl] * SCALE
        return carry

    lax.fori_loop(0, ROWS_PER_W, row_body, 0)

    pltpu.sync_copy(rows_v, out_hbm.at[pl.ds(base, ROWS_PER_W)])


@jax.jit
def _emb_call(x2d, pos, table):
    mesh = plsc.VectorSubcoreMesh(core_axis_name="c", subcore_axis_name="s")
    run = functools.partial(
        pl.kernel,
        mesh=mesh,
        out_type=jax.ShapeDtypeStruct((ROWS_TOTAL, D_MODEL), jnp.float32),
        scratch_types=[
            pltpu.VMEM((N_CHUNKS, GATHER_CHUNK), jnp.int32),
            pltpu.VMEM((ROWS_PER_W, D_MODEL), jnp.float32),
            pltpu.SemaphoreType.DMA,
        ],
    )(_emb_body)
    return run(x2d, pos, table)


def kernel(x, table):
    x2d = x.reshape(ROWS_TOTAL // GATHER_CHUNK, GATHER_CHUNK)
    out = _emb_call(x2d, _POS_OVER_SCALE, table)
    return out.reshape(BATCH, SEQ_LEN, D_MODEL)


# speedup vs baseline: 1.1198x; 1.0403x over previous
import functools
import math

import jax
import jax.numpy as jnp
import numpy as np
from jax import lax
from jax.experimental import pallas as pl
from jax.experimental.pallas import tpu as pltpu
from jax.experimental.pallas import tpu_sc as plsc

D_MODEL = 128
SEQ_LEN = 2048
BATCH = 4
SCALE = math.sqrt(float(D_MODEL))

NUM_CORES = 2
NUM_SUBCORES = 16
NUM_WORKERS = NUM_CORES * NUM_SUBCORES
POS_PER_W = SEQ_LEN // NUM_WORKERS
LANES = 16
CHUNKS = D_MODEL // LANES


def _positional_encoding() -> np.ndarray:
    half = D_MODEL / 2
    positions = np.arange(SEQ_LEN)[:, np.newaxis]
    depths = np.arange(int(half))[np.newaxis, :] / half
    angle_rates = 1 / 10000 ** depths
    angle_rads = positions * angle_rates
    pe = np.concatenate([np.sin(angle_rads), np.cos(angle_rads)], axis=-1)
    return pe.astype(np.float32)


_POS_ENC = jnp.asarray(_positional_encoding())


def _emb_body(x_hbm, pos_hbm, table_hbm, out_hbm, idx_v, pos_v, tile_v,
              gsem, wsem):
    wid = lax.axis_index("s") * NUM_CORES + lax.axis_index("c")
    pstart = wid * POS_PER_W

    pltpu.sync_copy(x_hbm.at[wid], idx_v)
    pltpu.sync_copy(pos_hbm.at[pl.ds(pstart, POS_PER_W)], pos_v)

    copies = []
    for j in range(2):
        copies.append(pltpu.async_copy(
            table_hbm.at[idx_v.at[j]],
            tile_v.at[pl.ds(j * 2 * POS_PER_W, 2 * POS_PER_W)],
            gsem))

    writes = []
    for j in range(2):
        copies[j].wait()
        for b in (2 * j, 2 * j + 1):

            def row_body(i, carry, b=b):
                r = b * POS_PER_W + i
                for c in range(CHUNKS):
                    sl = pl.ds(c * LANES, LANES)
                    tile_v[r, sl] = tile_v[r, sl] * SCALE + pos_v[i, sl]
                return carry

            lax.fori_loop(0, POS_PER_W, row_body, 0)
            writes.append(pltpu.async_copy(
                tile_v.at[pl.ds(b * POS_PER_W, POS_PER_W)],
                out_hbm.at[b, pl.ds(pstart, POS_PER_W)],
                wsem))

    for w in writes:
        w.wait()


@jax.jit
def _emb_call(x, pos, table):
    mesh = plsc.VectorSubcoreMesh(core_axis_name="c", subcore_axis_name="s")
    run = functools.partial(
        pl.kernel,
        mesh=mesh,
        out_type=jax.ShapeDtypeStruct((BATCH, SEQ_LEN, D_MODEL), jnp.float32),
        scratch_types=[
            pltpu.VMEM((2, 128), jnp.int32),
            pltpu.VMEM((POS_PER_W, D_MODEL), jnp.float32),
            pltpu.VMEM((BATCH * POS_PER_W, D_MODEL), jnp.float32),
            pltpu.SemaphoreType.DMA,
            pltpu.SemaphoreType.DMA,
        ],
    )(_emb_body)
    return run(x, pos, table)


def kernel(x, table):
    x_t = (x.reshape(BATCH, NUM_WORKERS, POS_PER_W)
            .transpose(1, 0, 2)
            .reshape(NUM_WORKERS, 2, 128))
    return _emb_call(x_t, _POS_ENC, table)


# speedup vs baseline: 1.1279x; 1.0072x over previous
import functools
import math

import jax
import jax.numpy as jnp
import numpy as np
from jax import lax
from jax.experimental import pallas as pl
from jax.experimental.pallas import tpu as pltpu
from jax.experimental.pallas import tpu_sc as plsc

D_MODEL = 128
SEQ_LEN = 2048
BATCH = 4
SCALE = math.sqrt(float(D_MODEL))

NUM_CORES = 2
NUM_SUBCORES = 16
NUM_WORKERS = NUM_CORES * NUM_SUBCORES
POS_PER_W = SEQ_LEN // NUM_WORKERS
X_ROWS = BATCH * SEQ_LEN // 128
LANES = 16
CHUNKS = D_MODEL // LANES


def _positional_encoding() -> np.ndarray:
    half = D_MODEL / 2
    positions = np.arange(SEQ_LEN)[:, np.newaxis]
    depths = np.arange(int(half))[np.newaxis, :] / half
    angle_rates = 1 / 10000 ** depths
    angle_rads = positions * angle_rates
    pe = np.concatenate([np.sin(angle_rads), np.cos(angle_rads)], axis=-1)
    return pe.astype(np.float32)


_POS_ENC = jnp.asarray(_positional_encoding())


def _emb_body(x_hbm, pos_hbm, table_hbm, out_hbm, idx_v, pos_v, tile_v,
              gsem, psem, wsem):
    wid = lax.axis_index("s") * NUM_CORES + lax.axis_index("c")
    pstart = wid * POS_PER_W
    w2 = wid // 2
    half = (wid % 2) * POS_PER_W

    for b in range(BATCH):
        pltpu.sync_copy(x_hbm.at[pl.ds(b * (X_ROWS // BATCH) + w2, 1)],
                        idx_v.at[pl.ds(b, 1)])

    copies = []
    for b in range(BATCH):
        copies.append(pltpu.async_copy(
            table_hbm.at[idx_v.at[b, pl.ds(half, POS_PER_W)]],
            tile_v.at[pl.ds(b * POS_PER_W, POS_PER_W)],
            gsem))

    pcopy = pltpu.async_copy(pos_hbm.at[pl.ds(pstart, POS_PER_W)], pos_v,
                             psem)
    pcopy.wait()

    writes = []
    for b in range(BATCH):
        copies[b].wait()

        def row_body(i, carry, b=b):
            r = b * POS_PER_W + i
            for c in range(CHUNKS):
                sl = pl.ds(c * LANES, LANES)
                tile_v[r, sl] = tile_v[r, sl] * SCALE + pos_v[i, sl]
            return carry

        lax.fori_loop(0, POS_PER_W, row_body, 0)
        writes.append(pltpu.async_copy(
            tile_v.at[pl.ds(b * POS_PER_W, POS_PER_W)],
            out_hbm.at[b, pl.ds(pstart, POS_PER_W)],
            wsem))

    for w in writes:
        w.wait()


@jax.jit
def _emb_call(x_r, pos, table):
    mesh = plsc.VectorSubcoreMesh(core_axis_name="c", subcore_axis_name="s")
    run = functools.partial(
        pl.kernel,
        mesh=mesh,
        out_type=jax.ShapeDtypeStruct((BATCH, SEQ_LEN, D_MODEL), jnp.float32),
        scratch_types=[
            pltpu.VMEM((BATCH, 128), jnp.int32),
            pltpu.VMEM((POS_PER_W, D_MODEL), jnp.float32),
            pltpu.VMEM((BATCH * POS_PER_W, D_MODEL), jnp.float32),
            pltpu.SemaphoreType.DMA,
            pltpu.SemaphoreType.DMA,
            pltpu.SemaphoreType.DMA,
        ],
    )(_emb_body)
    return run(x_r, pos, table)


def kernel(x, table):
    return _emb_call(x.reshape(X_ROWS, 128), _POS_ENC, table)


# speedup vs baseline: 1.1879x; 1.0532x over previous
import functools
import math

import jax
import jax.numpy as jnp
import numpy as np
from jax import lax
from jax.experimental import pallas as pl
from jax.experimental.pallas import tpu as pltpu
from jax.experimental.pallas import tpu_sc as plsc

D_MODEL = 128
SEQ_LEN = 2048
BATCH = 4
SCALE = math.sqrt(float(D_MODEL))

NUM_CORES = 2
NUM_SUBCORES = 16
NUM_WORKERS = NUM_CORES * NUM_SUBCORES
POS_PER_W = SEQ_LEN // NUM_WORKERS
LANES = 16
CHUNKS = D_MODEL // LANES


def _positional_encoding() -> np.ndarray:
    half = D_MODEL / 2
    positions = np.arange(SEQ_LEN)[:, np.newaxis]
    depths = np.arange(int(half))[np.newaxis, :] / half
    angle_rates = 1 / 10000 ** depths
    angle_rads = positions * angle_rates
    pe = np.concatenate([np.sin(angle_rads), np.cos(angle_rads)], axis=-1)
    return pe.astype(np.float32)


_POS_ENC_FLAT = jnp.asarray(_positional_encoding().reshape(-1))


def _emb_body(x_hbm, pos_hbm, table_hbm, out_hbm, idx_v, pos_v, tile_v,
              isem, gsem, psem, wsem):
    wid = lax.axis_index("s") * NUM_CORES + lax.axis_index("c")
    pstart = wid * POS_PER_W

    icopies = []
    for b in range(BATCH):
        icopies.append(pltpu.async_copy(
            x_hbm.at[pl.ds(b * SEQ_LEN + pstart, POS_PER_W)],
            idx_v.at[pl.ds(b * POS_PER_W, POS_PER_W)],
            isem))

    gcopies = []
    for b in range(BATCH):
        icopies[b].wait()
        gcopies.append(pltpu.async_copy(
            table_hbm.at[idx_v.at[pl.ds(b * POS_PER_W, POS_PER_W)]],
            tile_v.at[pl.ds(b * POS_PER_W, POS_PER_W)],
            gsem))

    pcopy = pltpu.async_copy(
        pos_hbm.at[pl.ds(pstart * D_MODEL, POS_PER_W * D_MODEL)], pos_v,
        psem)
    pcopy.wait()

    writes = []
    for b in range(BATCH):
        gcopies[b].wait()

        def row_body(i, carry, b=b):
            r = b * POS_PER_W + i
            for c in range(CHUNKS):
                tile_v[r, pl.ds(c * LANES, LANES)] = (
                    tile_v[r, pl.ds(c * LANES, LANES)] * SCALE
                    + pos_v[pl.ds(i * D_MODEL + c * LANES, LANES)])
            return carry

        lax.fori_loop(0, POS_PER_W, row_body, 0)
        writes.append(pltpu.async_copy(
            tile_v.at[pl.ds(b * POS_PER_W, POS_PER_W)],
            out_hbm.at[b, pl.ds(pstart, POS_PER_W)],
            wsem))

    for w in writes:
        w.wait()


@jax.jit
def _emb_call(x_flat, pos_flat, table):
    mesh = plsc.VectorSubcoreMesh(core_axis_name="c", subcore_axis_name="s")
    run = functools.partial(
        pl.kernel,
        mesh=mesh,
        out_type=jax.ShapeDtypeStruct((BATCH, SEQ_LEN, D_MODEL), jnp.float32),
        scratch_types=[
            pltpu.VMEM((BATCH * POS_PER_W,), jnp.int32),
            pltpu.VMEM((POS_PER_W * D_MODEL,), jnp.float32),
            pltpu.VMEM((BATCH * POS_PER_W, D_MODEL), jnp.float32),
            pltpu.SemaphoreType.DMA,
            pltpu.SemaphoreType.DMA,
            pltpu.SemaphoreType.DMA,
            pltpu.SemaphoreType.DMA,
        ],
    )(_emb_body)
    return run(x_flat, pos_flat, table)


def kernel(x, table):
    return _emb_call(x.reshape(-1), _POS_ENC_FLAT, table)
